# baseline (speedup 1.0000x reference)
import functools

import jax
import jax.numpy as jnp
from jax import lax
from jax.experimental import pallas as pl
from jax.experimental.pallas import tpu as pltpu
from jax.experimental.pallas import tpu_sc as plsc

B = 4096
ED = 32
NC = 2
NS = 16
NW = NC * NS
BPW = B // NW

_R = 256
_NBLK = B // _R


def _sc_gather_body(t0, t1, t2, t3, t4, t5, i0, i1, i2, i3, i4, i5,
                    o0, o1, o2, o3, o4, o5, idx_v, rows_v, sem):
    wid = lax.axis_index("s") * NC + lax.axis_index("c")
    base = wid * BPW
    for tab, idx, out in ((t0, i0, o0), (t1, i1, o1), (t2, i2, o2),
                          (t3, i3, o3), (t4, i4, o4), (t5, i5, o5)):
        pltpu.sync_copy(idx.at[pl.ds(base, BPW)], idx_v)
        pltpu.async_copy(tab.at[idx_v], rows_v, sem).wait()
        pltpu.sync_copy(rows_v, out.at[pl.ds(base, BPW)])


def _sc_gather(tables, idxs):
    mesh = plsc.VectorSubcoreMesh(core_axis_name="c", subcore_axis_name="s")
    fn = functools.partial(
        pl.kernel,
        mesh=mesh,
        compiler_params=pltpu.CompilerParams(use_tc_tiling_on_sc=False),
        out_type=[jax.ShapeDtypeStruct((B, ED), jnp.float32)] * 6,
        scratch_types=[
            pltpu.VMEM((BPW,), jnp.int32),
            pltpu.VMEM((BPW, ED), jnp.float32),
            pltpu.SemaphoreType.DMA,
        ],
    )(_sc_gather_body)
    return fn(*tables, *idxs)


def _tc_body(g0, g1, g2, g3, g4f, g5f, lab,
             Wu1, bu1, Wu2, bu2, Wi1, bi1, Wi2, bi2,
             out, item_scr):
    @pl.when(pl.program_id(0) == 0)
    def _():
        wi1 = Wi1[...]
        ih = jnp.maximum(
            g4f[...] @ wi1[:ED, :] + g5f[...] @ wi1[ED:, :] + bi1[...], 0.0)
        item_scr[...] = jnp.maximum(ih @ Wi2[...] + bi2[...], 0.0)

    wu1 = Wu1[...]
    uh = jnp.maximum(
        g0[...] @ wu1[0 * ED:1 * ED, :] + g1[...] @ wu1[1 * ED:2 * ED, :]
        + g2[...] @ wu1[2 * ED:3 * ED, :] + g3[...] @ wu1[3 * ED:4 * ED, :]
        + bu1[...], 0.0)
    uo = jnp.maximum(uh @ Wu2[...] + bu2[...], 0.0)
    logits = lax.dot_general(uo, item_scr[...],
                             (((1,), (1,)), ((), ())))
    m = jnp.max(logits, axis=1, keepdims=True)
    s = jnp.sum(jnp.exp(logits - m), axis=1, keepdims=True)
    lse = m + jnp.log(s)
    cols = lax.broadcasted_iota(jnp.int32, (_R, B), 1)
    tgt = jnp.sum(jnp.where(cols == lab[...], logits, 0.0), axis=1,
                  keepdims=True)
    out[...] = lse - tgt


def _tc_dense(g, labels, Wu1, bu1, Wu2, bu2, Wi1, bi1, Wi2, bi2):
    blk = lambda i: (i, 0)
    full = lambda i: (0, 0)
    return pl.pallas_call(
        _tc_body,
        grid=(_NBLK,),
        in_specs=[
            pl.BlockSpec((_R, ED), blk), pl.BlockSpec((_R, ED), blk),
            pl.BlockSpec((_R, ED), blk), pl.BlockSpec((_R, ED), blk),
            pl.BlockSpec((B, ED), full), pl.BlockSpec((B, ED), full),
            pl.BlockSpec((_R, 1), blk),
            pl.BlockSpec((128, 64), full), pl.BlockSpec((1, 64), full),
            pl.BlockSpec((64, 32), full), pl.BlockSpec((1, 32), full),
            pl.BlockSpec((64, 64), full), pl.BlockSpec((1, 64), full),
            pl.BlockSpec((64, 32), full), pl.BlockSpec((1, 32), full),
        ],
        out_specs=pl.BlockSpec((_R, 1), blk),
        out_shape=jax.ShapeDtypeStruct((B, 1), jnp.float32),
        scratch_shapes=[pltpu.VMEM((B, ED), jnp.float32)],
    )(g[0], g[1], g[2], g[3], g[4], g[5], labels,
      Wu1, bu1, Wu2, bu2, Wi1, bi1, Wi2, bi2)


def kernel(user_id, user_city, user_device, user_age, item_id, item_cate,
           labels, E_user_id, E_user_city, E_user_device, E_user_age,
           E_item_id, E_item_cate, Wu1, bu1, Wu2, bu2, Wi1, bi1, Wi2, bi2):
    idxs = [x[:, 0].astype(jnp.int32) for x in
            (user_id, user_city, user_device, user_age, item_id, item_cate)]
    tables = (E_user_id, E_user_city, E_user_device, E_user_age,
              E_item_id, E_item_cate)
    g = _sc_gather(tables, idxs)
    loss = _tc_dense(g, labels.astype(jnp.int32),
                     Wu1, bu1.reshape(1, -1), Wu2, bu2.reshape(1, -1),
                     Wi1, bi1.reshape(1, -1), Wi2, bi2.reshape(1, -1))
    return loss[:, 0]


# speedup vs baseline: 2.0316x; 2.0316x over previous
import functools

import jax
import jax.numpy as jnp
from jax import lax
from jax.experimental import pallas as pl
from jax.experimental.pallas import tpu as pltpu
from jax.experimental.pallas import tpu_sc as plsc

B = 4096
ED = 32
NC = 2
NS = 16
NW = NC * NS
BPW = B // NW
_C = 32

_R = 256
_NBLK = B // _R


def _sc_gather_body(t0, t1, t2, t3, t4, t5, i0, i1, i2, i3, i4, i5,
                    o0, o1, o2, o3, o4, o5,
                    idx_v, buf_v, rows_v, sem):
    wid = lax.axis_index("s") * NC + lax.axis_index("c")
    base = wid * BPW
    for tab, idx, out in ((t0, i0, o0), (t1, i1, o1), (t2, i2, o2),
                          (t3, i3, o3), (t4, i4, o4), (t5, i5, o5)):
        pltpu.sync_copy(idx.at[pl.ds(base, BPW)], idx_v)

        for c in range(BPW // _C):
            def fire(k, _):
                v = idx_v[pl.ds(c * _C + k * 16, 16)]
                for j in range(16):
                    t = lax.shift_right_logical(v[j], 3)
                    pltpu.async_copy(tab.at[pl.ds(t, 1)],
                                     buf_v.at[pl.ds(k * 16 + j, 1)], sem)
                return 0

            lax.fori_loop(0, _C // 16, fire, 0)
            pltpu.make_async_copy(tab.at[pl.ds(0, _C)], buf_v, sem).wait()

            def row(k, _):
                v = idx_v[pl.ds(c * _C + k * 16, 16)]
                for j in range(16):
                    s = v[j] & 7
                    r = k * 16 + j
                    rows_v[c * _C + r, pl.ds(0, 16)] = (
                        buf_v[r, s, pl.ds(0, 16)])
                    rows_v[c * _C + r, pl.ds(16, 16)] = (
                        buf_v[r, s, pl.ds(16, 16)])
                return 0

            lax.fori_loop(0, _C // 16, row, 0)
        pltpu.sync_copy(rows_v, out.at[pl.ds(base, BPW)])


def _sc_gather(tables, idxs):
    mesh = plsc.VectorSubcoreMesh(core_axis_name="c", subcore_axis_name="s")
    fn = functools.partial(
        pl.kernel,
        mesh=mesh,
        out_type=[jax.ShapeDtypeStruct((B, ED), jnp.float32)] * 6,
        scratch_types=[
            pltpu.VMEM((BPW,), jnp.int32),
            pltpu.VMEM((_C, 8, ED), jnp.float32),
            pltpu.VMEM((BPW, ED), jnp.float32),
            pltpu.SemaphoreType.DMA,
        ],
    )(_sc_gather_body)
    return fn(*tables, *idxs)


def _tc_body(g0, g1, g2, g3, g4f, g5f, lab,
             Wu1, bu1, Wu2, bu2, Wi1, bi1, Wi2, bi2,
             out, item_scr):
    @pl.when(pl.program_id(0) == 0)
    def _():
        wi1 = Wi1[...]
        ih = jnp.maximum(
            g4f[...] @ wi1[:ED, :] + g5f[...] @ wi1[ED:, :] + bi1[...], 0.0)
        item_scr[...] = jnp.maximum(ih @ Wi2[...] + bi2[...], 0.0)

    wu1 = Wu1[...]
    uh = jnp.maximum(
        g0[...] @ wu1[0 * ED:1 * ED, :] + g1[...] @ wu1[1 * ED:2 * ED, :]
        + g2[...] @ wu1[2 * ED:3 * ED, :] + g3[...] @ wu1[3 * ED:4 * ED, :]
        + bu1[...], 0.0)
    uo = jnp.maximum(uh @ Wu2[...] + bu2[...], 0.0)
    logits = lax.dot_general(uo, item_scr[...],
                             (((1,), (1,)), ((), ())))
    m = jnp.max(logits, axis=1, keepdims=True)
    s = jnp.sum(jnp.exp(logits - m), axis=1, keepdims=True)
    lse = m + jnp.log(s)
    cols = lax.broadcasted_iota(jnp.int32, (_R, B), 1)
    tgt = jnp.sum(jnp.where(cols == lab[...], logits, 0.0), axis=1,
                  keepdims=True)
    out[...] = lse - tgt


def _tc_dense(g, labels, Wu1, bu1, Wu2, bu2, Wi1, bi1, Wi2, bi2):
    blk = lambda i: (i, 0)
    full = lambda i: (0, 0)
    return pl.pallas_call(
        _tc_body,
        grid=(_NBLK,),
        in_specs=[
            pl.BlockSpec((_R, ED), blk), pl.BlockSpec((_R, ED), blk),
            pl.BlockSpec((_R, ED), blk), pl.BlockSpec((_R, ED), blk),
            pl.BlockSpec((B, ED), full), pl.BlockSpec((B, ED), full),
            pl.BlockSpec((_R, 1), blk),
            pl.BlockSpec((128, 64), full), pl.BlockSpec((1, 64), full),
            pl.BlockSpec((64, 32), full), pl.BlockSpec((1, 32), full),
            pl.BlockSpec((64, 64), full), pl.BlockSpec((1, 64), full),
            pl.BlockSpec((64, 32), full), pl.BlockSpec((1, 32), full),
        ],
        out_specs=pl.BlockSpec((_R, 1), blk),
        out_shape=jax.ShapeDtypeStruct((B, 1), jnp.float32),
        scratch_shapes=[pltpu.VMEM((B, ED), jnp.float32)],
    )(g[0], g[1], g[2], g[3], g[4], g[5], labels,
      Wu1, bu1, Wu2, bu2, Wi1, bi1, Wi2, bi2)


def kernel(user_id, user_city, user_device, user_age, item_id, item_cate,
           labels, E_user_id, E_user_city, E_user_device, E_user_age,
           E_item_id, E_item_cate, Wu1, bu1, Wu2, bu2, Wi1, bi1, Wi2, bi2):
    idxs = [x[:, 0].astype(jnp.int32) for x in
            (user_id, user_city, user_device, user_age, item_id, item_cate)]
    E_user_age_p = jnp.concatenate(
        [E_user_age, jnp.zeros((4, ED), E_user_age.dtype)], axis=0)
    tables = tuple(t.reshape(t.shape[0] // 8, 8, ED) for t in
                   (E_user_id, E_user_city, E_user_device, E_user_age_p,
                    E_item_id, E_item_cate))
    g = _sc_gather(tables, idxs)
    loss = _tc_dense(g, labels.astype(jnp.int32),
                     Wu1, bu1.reshape(1, -1), Wu2, bu2.reshape(1, -1),
                     Wi1, bi1.reshape(1, -1), Wi2, bi2.reshape(1, -1))
    return loss[:, 0]
